# baseline (speedup 1.0000x reference)
import functools

import jax
import jax.numpy as jnp
from jax import lax
from jax.experimental import pallas as pl
from jax.experimental.pallas import tpu as pltpu
from jax.experimental.pallas import tpu_sc as plsc

_NUM_SAMPLES = 512
_N_BISECT = 26
_BB = 8
_BD = 256
_L = 16


_NUM_SC_CORES = 2
_NUM_SC_SUBCORES = 16


def _make_sc_stats(b, d):
    nc = _NUM_SC_CORES
    n_chunks = d // _L
    rank = d - d * 3 // 4
    mesh = plsc.VectorSubcoreMesh(
        core_axis_name="c", subcore_axis_name="s",
        num_cores=_NUM_SC_CORES, num_subcores=_NUM_SC_SUBCORES)

    @functools.partial(
        pl.kernel,
        out_type=jax.ShapeDtypeStruct((b, _L), jnp.float32),
        mesh=mesh,
        scratch_types=[
            pltpu.VMEM((d,), jnp.float32),
            pltpu.VMEM((_L,), jnp.float32),
        ],
        compiler_params=pltpu.CompilerParams(
            needs_layout_passes=False, skip_device_barrier=True),
    )
    def sc_stats(x_hbm, out_hbm, row_v, out_v):
        wid = lax.axis_index("s") * nc + lax.axis_index("c")
        pltpu.sync_copy(x_hbm.at[wid], row_v)

        def chunk(i):
            return row_v[pl.ds(i * _L, _L)]

        zeros = jnp.zeros((_L,), jnp.float32)

        def reduce_scalar(vec, op):
            acc = vec[0]
            for i in range(1, _L):
                acc = op(acc, vec[i])
            return acc

        c0 = chunk(0)
        s_acc, mn_acc, mx_acc = c0, c0, c0
        for i in range(1, n_chunks):
            c = chunk(i)
            s_acc = s_acc + c
            mn_acc = jnp.minimum(mn_acc, c)
            mx_acc = jnp.maximum(mx_acc, c)
        total = reduce_scalar(s_acc, lax.add)
        mu = total * jnp.float32(1.0 / d)
        lo0 = reduce_scalar(mn_acc, lax.min)
        hi0 = reduce_scalar(mx_acc, lax.max)
        mu_vec = jnp.full((_L,), mu, jnp.float32)

        sq_acc = zeros
        for i in range(n_chunks):
            t = chunk(i) - mu_vec
            sq_acc = sq_acc + t * t
        sumsq = reduce_scalar(sq_acc, lax.add)

        rank_i = jnp.int32(rank)
        izeros = jnp.zeros((_L,), jnp.int32)

        def bisect(_, carry):
            lo, hi = carry
            mid = lo * 0.5 + hi * 0.5
            mid_vec = jnp.full((_L,), mid, jnp.float32)
            accs = [izeros, izeros, izeros, izeros]
            for i in range(n_chunks):
                accs[i % 4] = accs[i % 4] + plsc.all_reduce_population_count(
                    chunk(i) <= mid_vec)
            cnt = ((accs[0] + accs[1]) + (accs[2] + accs[3]))[0]
            pred = cnt >= rank_i
            return jnp.where(pred, lo, mid), jnp.where(pred, mid, hi)

        _, mean = lax.fori_loop(0, _N_BISECT, bisect, (lo0, hi0))

        lane = lax.iota(jnp.int32, _L)
        res = jnp.where(lane == 0, jnp.full((_L,), mean, jnp.float32),
                        jnp.where(lane == 1, mu_vec,
                                  jnp.full((_L,), sumsq, jnp.float32)))
        out_v[...] = res
        pltpu.sync_copy(out_v, out_hbm.at[wid])

    return sc_stats


def _bcast_kernel(d, x_ref, st_ref, o_ref):
    x = x_ref[...]
    st = st_ref[...]
    mean = st[:, 0:1]
    mu = st[:, 1:2]
    sumsq = st[:, 2:3]
    del mu
    std = jnp.sqrt(sumsq / (d - 1))
    x_norm = (x - mean) / std ** 0.3
    expo = jnp.clip(-x_norm / 0.001, -50.0, 50.0)
    y = 1.0 / (1.0 + jnp.exp(expo))
    o_ref[...] = jnp.broadcast_to(y[..., None], o_ref.shape)


def kernel(x, k):
    del k
    b, d = x.shape

    stats = _make_sc_stats(b, d)(x)

    out = pl.pallas_call(
        functools.partial(_bcast_kernel, d),
        grid=(b // _BB, d // _BD),
        in_specs=[
            pl.BlockSpec((_BB, _BD), lambda i, j: (i, j)),
            pl.BlockSpec((_BB, _L), lambda i, j: (i, 0)),
        ],
        out_specs=pl.BlockSpec((_BB, _BD, _NUM_SAMPLES), lambda i, j: (i, j, 0)),
        out_shape=jax.ShapeDtypeStruct((b, d, _NUM_SAMPLES), x.dtype),
    )(x, stats)
    return out


# speedup vs baseline: 1.0050x; 1.0050x over previous
import functools

import jax
import jax.numpy as jnp
from jax import lax
from jax.experimental import pallas as pl
from jax.experimental.pallas import tpu as pltpu
from jax.experimental.pallas import tpu_sc as plsc

_NUM_SAMPLES = 512
_N_BISECT = 26
_BB = 8
_BD = 256
_L = 16


_NUM_SC_CORES = 2
_NUM_SC_SUBCORES = 16


def _make_sc_stats(b, d):
    nc = _NUM_SC_CORES
    n_chunks = d // _L
    rank = d - d * 3 // 4
    mesh = plsc.VectorSubcoreMesh(
        core_axis_name="c", subcore_axis_name="s",
        num_cores=_NUM_SC_CORES, num_subcores=_NUM_SC_SUBCORES)

    @functools.partial(
        pl.kernel,
        out_type=jax.ShapeDtypeStruct((b, _L), jnp.float32),
        mesh=mesh,
        scratch_types=[
            pltpu.VMEM((d,), jnp.float32),
            pltpu.VMEM((_L,), jnp.float32),
        ],
        compiler_params=pltpu.CompilerParams(needs_layout_passes=False),
    )
    def sc_stats(x_hbm, out_hbm, row_v, out_v):
        wid = lax.axis_index("s") * nc + lax.axis_index("c")
        pltpu.sync_copy(x_hbm.at[wid], row_v)

        def chunk(i):
            return row_v[pl.ds(i * _L, _L)]

        zeros = jnp.zeros((_L,), jnp.float32)

        def reduce_scalar(vec, op):
            acc = vec[0]
            for i in range(1, _L):
                acc = op(acc, vec[i])
            return acc

        c0 = chunk(0)
        s_acc, mn_acc, mx_acc = c0, c0, c0
        for i in range(1, n_chunks):
            c = chunk(i)
            s_acc = s_acc + c
            mn_acc = jnp.minimum(mn_acc, c)
            mx_acc = jnp.maximum(mx_acc, c)
        total = reduce_scalar(s_acc, lax.add)
        mu = total * jnp.float32(1.0 / d)
        lo0 = reduce_scalar(mn_acc, lax.min)
        hi0 = reduce_scalar(mx_acc, lax.max)
        mu_vec = jnp.full((_L,), mu, jnp.float32)

        sq_acc = zeros
        for i in range(n_chunks):
            t = chunk(i) - mu_vec
            sq_acc = sq_acc + t * t
        sumsq = reduce_scalar(sq_acc, lax.add)

        rank_i = jnp.int32(rank)
        izeros = jnp.zeros((_L,), jnp.int32)

        def bisect(_, carry):
            lo, hi = carry
            mid = lo * 0.5 + hi * 0.5
            mid_vec = jnp.full((_L,), mid, jnp.float32)
            accs = [izeros, izeros, izeros, izeros]
            for i in range(n_chunks):
                accs[i % 4] = accs[i % 4] + plsc.all_reduce_population_count(
                    chunk(i) <= mid_vec)
            cnt = ((accs[0] + accs[1]) + (accs[2] + accs[3]))[0]
            pred = cnt >= rank_i
            return jnp.where(pred, lo, mid), jnp.where(pred, mid, hi)

        _, mean = lax.fori_loop(0, _N_BISECT, bisect, (lo0, hi0))

        lane = lax.iota(jnp.int32, _L)
        res = jnp.where(lane == 0, jnp.full((_L,), mean, jnp.float32),
                        jnp.where(lane == 1, mu_vec,
                                  jnp.full((_L,), sumsq, jnp.float32)))
        out_v[...] = res
        pltpu.sync_copy(out_v, out_hbm.at[wid])

    return sc_stats


def _bcast_kernel(d, x_ref, st_ref, o_ref):
    x = x_ref[...]
    st = st_ref[...]
    mean = st[:, 0:1]
    mu = st[:, 1:2]
    sumsq = st[:, 2:3]
    del mu
    std = jnp.sqrt(sumsq / (d - 1))
    x_norm = (x - mean) / std ** 0.3
    expo = jnp.clip(-x_norm / 0.001, -50.0, 50.0)
    y = 1.0 / (1.0 + jnp.exp(expo))
    o_ref[...] = jnp.broadcast_to(y[..., None], o_ref.shape)


def kernel(x, k):
    del k
    b, d = x.shape

    stats = _make_sc_stats(b, d)(x)

    out = pl.pallas_call(
        functools.partial(_bcast_kernel, d),
        grid=(b // _BB, d // _BD),
        in_specs=[
            pl.BlockSpec((_BB, _BD), lambda i, j: (i, j)),
            pl.BlockSpec((_BB, _L), lambda i, j: (i, 0)),
        ],
        out_specs=pl.BlockSpec((_BB, _BD, _NUM_SAMPLES), lambda i, j: (i, j, 0)),
        out_shape=jax.ShapeDtypeStruct((b, d, _NUM_SAMPLES), x.dtype),
    )(x, stats)
    return out
